# baseline (speedup 1.0000x reference)
import functools

import jax
import jax.numpy as jnp
from jax import lax
from jax.experimental import pallas as pl
from jax.experimental.pallas import tpu as pltpu
from jax.experimental.pallas import tpu_sc as plsc

BATCH = 4096
HIST = 200
D = 64
V = 1000000
NC = 2
NS = 16
NW = NC * NS
VP = 128
NBLK = V // VP
TAIL = V - NBLK * VP
UPW = NBLK // NW + 1
BB = BATCH // NW

_mesh = plsc.VectorSubcoreMesh(core_axis_name="c", subcore_axis_name="s")
_params = pltpu.CompilerParams(
    use_tc_tiling_on_sc=True, needs_layout_passes=False
)


def _iota16():
    return lax.iota(jnp.int32, 16)


@functools.partial(
    pl.kernel,
    mesh=_mesh,
    out_type=jax.ShapeDtypeStruct((V, VP), jnp.float32),
    scratch_types=[
        pltpu.VMEM((2, D, VP), jnp.float32),
        pltpu.VMEM((2, VP, VP), jnp.float32),
        pltpu.VMEM((D, TAIL), jnp.float32),
        pltpu.SemaphoreType.DMA,
        pltpu.SemaphoreType.DMA,
        pltpu.SemaphoreType.DMA,
        pltpu.SemaphoreType.DMA,
    ],
    compiler_params=_params,
)
def _table_rows(tt_hbm, tp_hbm, sbuf, obuf, tsb, ssem0, ssem1, wsem0, wsem1):
    cid = lax.axis_index("c")
    sid = lax.axis_index("s")
    wid = sid * NC + cid

    def transpose_block(src, b, width):
        def crow(c, _):
            for g in range(D // 16):
                v = plsc.load_gather(
                    src, [_iota16() + 16 * g, jnp.full((16,), c, jnp.int32)]
                )
                obuf[b, c, pl.ds(16 * g, 16)] = v
            return 0
        lax.fori_loop(0, width, crow, 0)

    @pl.when(wid == 0)
    def _():
        pltpu.sync_copy(tt_hbm.at[:, pl.ds(NBLK * VP, TAIL)], tsb)
        transpose_block(tsb, 0, TAIL)
        pltpu.sync_copy(obuf.at[0, pl.ds(0, TAIL)], tp_hbm.at[pl.ds(NBLK * VP, TAIL), :])

    def blk_of(u):
        return jnp.minimum(wid * UPW + u, NBLK - 1)

    def c0_of(u):
        return pl.multiple_of(blk_of(u) * VP, VP)

    def fire_stage(u, b, sem):
        pltpu.async_copy(tt_hbm.at[:, pl.ds(c0_of(u), VP)], sbuf.at[b], sem)

    def drain_stage(b, sem):
        pltpu.make_async_copy(
            tt_hbm.at[:, pl.ds(0, VP)], sbuf.at[b], sem
        ).wait()

    def wait_write(b, sem):
        pltpu.make_async_copy(
            obuf.at[b], tp_hbm.at[pl.ds(0, VP), :], sem
        ).wait()

    fire_stage(0, 0, ssem0)

    def pair(p, _):
        fire_stage(2 * p + 1, 1, ssem1)
        drain_stage(0, ssem0)

        @pl.when(p >= 1)
        def _():
            wait_write(0, wsem0)
        transpose_block(sbuf.at[0], 0, VP)
        pltpu.async_copy(
            obuf.at[0], tp_hbm.at[pl.ds(c0_of(2 * p), VP), :], wsem0
        )

        fire_stage(2 * p + 2, 0, ssem0)
        drain_stage(1, ssem1)

        @pl.when(p >= 1)
        def _():
            wait_write(1, wsem1)
        transpose_block(sbuf.at[1], 1, VP)
        pltpu.async_copy(
            obuf.at[1], tp_hbm.at[pl.ds(c0_of(2 * p + 1), VP), :], wsem1
        )
        return 0

    lax.fori_loop(0, UPW // 2, pair, 0)
    drain_stage(0, ssem0)
    wait_write(0, wsem0)
    transpose_block(sbuf.at[0], 0, VP)
    pltpu.async_copy(
        obuf.at[0], tp_hbm.at[pl.ds(c0_of(UPW - 1), VP), :], wsem0
    )
    wait_write(0, wsem0)
    wait_write(1, wsem1)


@functools.partial(
    pl.kernel,
    mesh=_mesh,
    out_type=jax.ShapeDtypeStruct((HIST, D, BATCH), jnp.float32),
    scratch_types=[
        pltpu.VMEM((HIST, BB), jnp.int32),
        pltpu.VMEM((2, BB, VP), jnp.float32),
        pltpu.VMEM((2, D, BB), jnp.float32),
        pltpu.SemaphoreType.DMA,
        pltpu.SemaphoreType.DMA,
        pltpu.SemaphoreType.DMA,
        pltpu.SemaphoreType.DMA,
    ],
    compiler_params=_params,
)
def _gather_t(it_hbm, tp_hbm, out_hbm, idx_v, gbuf, obuf,
              gsem0, gsem1, wsem0, wsem1):
    cid = lax.axis_index("c")
    sid = lax.axis_index("s")
    wid = sid * NC + cid
    b0 = pl.multiple_of(wid * BB, BB)
    pltpu.sync_copy(it_hbm.at[:, pl.ds(b0, BB)], idx_v)

    def fire_gather(t, b, sem):
        pltpu.async_copy(tp_hbm.at[idx_v.at[t]], gbuf.at[b], sem)

    def drain_gather(b, sem):
        pltpu.make_async_copy(
            tp_hbm.at[idx_v.at[0]], gbuf.at[b], sem
        ).wait()

    def wait_write(b, sem):
        pltpu.make_async_copy(
            obuf.at[b], out_hbm.at[0, :, pl.ds(b0, BB)], sem
        ).wait()

    def transpose_block(b):
        def drow(e, _):
            for g in range(BB // 16):
                v = plsc.load_gather(
                    gbuf.at[b],
                    [_iota16() + 16 * g, jnp.full((16,), e, jnp.int32)],
                )
                obuf[b, e, pl.ds(16 * g, 16)] = v
            return 0
        lax.fori_loop(0, D, drow, 0)

    fire_gather(0, 0, gsem0)

    def pair(p, _):
        fire_gather(2 * p + 1, 1, gsem1)
        drain_gather(0, gsem0)

        @pl.when(p >= 1)
        def _():
            wait_write(0, wsem0)
        transpose_block(0)
        pltpu.async_copy(
            obuf.at[0], out_hbm.at[2 * p, :, pl.ds(b0, BB)], wsem0
        )

        @pl.when(p < HIST // 2 - 1)
        def _():
            fire_gather(2 * p + 2, 0, gsem0)
        drain_gather(1, gsem1)

        @pl.when(p >= 1)
        def _():
            wait_write(1, wsem1)
        transpose_block(1)
        pltpu.async_copy(
            obuf.at[1], out_hbm.at[2 * p + 1, :, pl.ds(b0, BB)], wsem1
        )
        return 0

    lax.fori_loop(0, HIST // 2, pair, 0)
    wait_write(0, wsem0)
    wait_write(1, wsem1)


def kernel(items, table):
    items_t = items.astype(jnp.int32).T
    table_t = table.T
    tp = _table_rows(table_t)
    out_t = _gather_t(items_t, tp)
    return jnp.transpose(out_t, (2, 0, 1))


# speedup vs baseline: 1.7760x; 1.7760x over previous
import functools

import jax
import jax.numpy as jnp
from jax import lax
from jax.experimental import pallas as pl
from jax.experimental.pallas import tpu as pltpu
from jax.experimental.pallas import tpu_sc as plsc

BATCH = 4096
HIST = 200
D = 64
V = 1000000
NC = 2
NS = 16
NW = NC * NS
VP = 128
BB = BATCH // NW

_mesh = plsc.VectorSubcoreMesh(core_axis_name="c", subcore_axis_name="s")
_params = pltpu.CompilerParams(
    use_tc_tiling_on_sc=True, needs_layout_passes=False
)


@functools.partial(
    pl.kernel,
    mesh=_mesh,
    out_type=jax.ShapeDtypeStruct((HIST, D, BATCH), jnp.float32),
    scratch_types=[
        pltpu.VMEM((HIST, BB), jnp.int32),
        pltpu.VMEM((2, BB, VP), jnp.float32),
        pltpu.VMEM((2, D, BB), jnp.float32),
        pltpu.SemaphoreType.DMA,
        pltpu.SemaphoreType.DMA,
        pltpu.SemaphoreType.DMA,
        pltpu.SemaphoreType.DMA,
    ],
    compiler_params=_params,
)
def _gather_t(it_hbm, tp_hbm, out_hbm, idx_v, gbuf, obuf,
              gsem0, gsem1, wsem0, wsem1):
    cid = lax.axis_index("c")
    sid = lax.axis_index("s")
    wid = sid * NC + cid
    b0 = pl.multiple_of(wid * BB, BB)
    pltpu.sync_copy(it_hbm.at[:, pl.ds(b0, BB)], idx_v)

    rows = [lax.iota(jnp.int32, 16) + 16 * g for g in range(BB // 16)]

    def fire_gather(t, b, sem):
        pltpu.async_copy(tp_hbm.at[idx_v.at[t]], gbuf.at[b], sem)

    def drain_gather(b, sem):
        pltpu.make_async_copy(
            tp_hbm.at[idx_v.at[0]], gbuf.at[b], sem
        ).wait()

    def wait_write(b, sem):
        pltpu.make_async_copy(
            obuf.at[b], out_hbm.at[0, :, pl.ds(b0, BB)], sem
        ).wait()

    def transpose_block(b):
        def drow(eo, _):
            for ei in range(8):
                e = eo * 8 + ei
                cols = jnp.full((16,), e, jnp.int32)
                vals = [
                    plsc.load_gather(gbuf.at[b], [rows[g], cols])
                    for g in range(BB // 16)
                ]
                for g in range(BB // 16):
                    obuf[b, e, pl.ds(16 * g, 16)] = vals[g]
            return 0
        lax.fori_loop(0, D // 8, drow, 0)

    fire_gather(0, 0, gsem0)

    def pair(p, _):
        fire_gather(2 * p + 1, 1, gsem1)
        drain_gather(0, gsem0)

        @pl.when(p >= 1)
        def _():
            wait_write(0, wsem0)
        transpose_block(0)
        pltpu.async_copy(
            obuf.at[0], out_hbm.at[2 * p, :, pl.ds(b0, BB)], wsem0
        )

        @pl.when(p < HIST // 2 - 1)
        def _():
            fire_gather(2 * p + 2, 0, gsem0)
        drain_gather(1, gsem1)

        @pl.when(p >= 1)
        def _():
            wait_write(1, wsem1)
        transpose_block(1)
        pltpu.async_copy(
            obuf.at[1], out_hbm.at[2 * p + 1, :, pl.ds(b0, BB)], wsem1
        )
        return 0

    lax.fori_loop(0, HIST // 2, pair, 0)
    wait_write(0, wsem0)
    wait_write(1, wsem1)


def kernel(items, table):
    items_t = items.astype(jnp.int32).T
    tp = jnp.pad(table, ((0, 0), (0, VP - D)))
    out_t = _gather_t(items_t, tp)
    return jnp.transpose(out_t, (2, 0, 1))
